# baseline (speedup 1.0000x reference)
import jax
import jax.numpy as jnp
from jax.experimental import pallas as pl
from jax.experimental.pallas import tpu as pltpu

N = 100000
C = 128
T = 4
EPS = 1e-05
BLK_S = 10000
BLK_N = 5000
NS_S = N // BLK_S
NS_N = N // BLK_N


def _stats_body(sb, pq, pv, nb, out, acc):
    i = pl.program_id(0)

    @pl.when(i == 0)
    def _init():
        acc[...] = jnp.zeros_like(acc)

    srows, qrows = [], []
    for ref in (sb, pq, pv, nb):
        x = ref[...]
        xr = x.reshape(BLK_S // 8, 8, C)
        srows.append(jnp.sum(xr, axis=0))
        qrows.append(jnp.sum(xr * xr, axis=0))
    acc[...] += jnp.stack(srows + qrows, axis=0)

    @pl.when(i == NS_S - 1)
    def _fin():
        out[...] = jnp.sum(acc[...], axis=1)


def _norm_body(stats, w, b, sb, pq, pv, nb, osb, opq, opv, onb, ss):
    i = pl.program_id(0)

    @pl.when(i == 0)
    def _scale():
        tot = stats[...]
        inv_n = 1.0 / N
        mean = tot[:T, :] * inv_n
        var = tot[T:, :] * inv_n - mean * mean
        inv_std = jax.lax.rsqrt(jnp.clip(var, EPS, None))
        scale = w[...] * inv_std
        shift = b[...] - mean * scale
        ss[...] = jnp.concatenate([scale, shift], axis=0)

    for t, (ref, oref) in enumerate(((sb, osb), (pq, opq), (pv, opv), (nb, onb))):
        oref[...] = ref[...] * ss[t:t + 1, :] + ss[T + t:T + t + 1, :]


@jax.jit
def kernel(SB, PQ, PV, NB, weight, bias):
    stats = pl.pallas_call(
        _stats_body,
        grid=(NS_S,),
        in_specs=[pl.BlockSpec((BLK_S, C), lambda i: (i, 0))] * 4,
        out_specs=pl.BlockSpec((2 * T, C), lambda i: (0, 0)),
        out_shape=jax.ShapeDtypeStruct((2 * T, C), jnp.float32),
        scratch_shapes=[pltpu.VMEM((2 * T, 8, C), jnp.float32)],
    )(SB, PQ, PV, NB)

    data_spec = pl.BlockSpec((BLK_N, C), lambda i: (i, 0))
    const_spec = pl.BlockSpec((2 * T, C), lambda i: (0, 0))
    wb_spec = pl.BlockSpec((T, C), lambda i: (0, 0))
    outs = pl.pallas_call(
        _norm_body,
        grid=(NS_N,),
        in_specs=[const_spec, wb_spec, wb_spec] + [data_spec] * 4,
        out_specs=[data_spec] * 4,
        out_shape=[jax.ShapeDtypeStruct((N, C), jnp.float32)] * 4,
        scratch_shapes=[pltpu.VMEM((2 * T, C), jnp.float32)],
    )(stats, weight, bias, SB, PQ, PV, NB)
    return tuple(outs)
